# baseline (speedup 1.0000x reference)
import jax
import jax.numpy as jnp
from jax.experimental import pallas as pl
from jax.experimental.pallas import tpu as pltpu


def _qsample_body(t_ref, c1tab_ref, c2tab_ref, x_ref, n_ref, o_ref):
    b = pl.program_id(0)
    tt = t_ref[b]
    c1 = c1tab_ref[tt]
    c2 = c2tab_ref[tt]
    o_ref[...] = x_ref[...] * c1 + n_ref[...] * c2


def kernel(x_start, t, noise, sqrt_alphas_cumprod, sqrt_one_minus_alphas_cumprod):
    B, C, H, W = x_start.shape
    HW = H * W
    x3 = x_start.reshape(B, C, HW)
    n3 = noise.reshape(B, C, HW)
    CB = 8
    grid = (B, C // CB)

    data_spec = pl.BlockSpec((1, CB, HW), lambda b, c, *_: (b, c, 0))
    out = pl.pallas_call(
        _qsample_body,
        grid_spec=pltpu.PrefetchScalarGridSpec(
            num_scalar_prefetch=3,
            grid=grid,
            in_specs=[data_spec, data_spec],
            out_specs=data_spec,
        ),
        out_shape=jax.ShapeDtypeStruct((B, C, HW), x_start.dtype),
    )(t, sqrt_alphas_cumprod, sqrt_one_minus_alphas_cumprod, x3, n3)
    return out.reshape(B, C, H, W)
